# baseline (speedup 1.0000x reference)
import jax
import jax.numpy as jnp
from jax.experimental import pallas as pl
from jax.experimental.pallas import tpu as pltpu

_B, _C, _L, _D = 4096, 64, 128, 256
_CL = _C * _L
_EPS = 1e-8

_BM = 1024
_CHUNK = 2048


def _normalize_body(e_ref, out_ref):
    e = e_ref[...]
    norm = jnp.sqrt(jnp.sum(e * e, axis=1, keepdims=True))
    inv = 1.0 / jnp.maximum(norm, _EPS)
    out_ref[...] = (e * inv).astype(jnp.bfloat16)


def _main_body(x_ref, e_hbm, out_ref, e_vmem, sem):
    @pl.when(pl.program_id(0) == 0)
    def _():
        cp = pltpu.make_async_copy(e_hbm, e_vmem, sem)
        cp.start()
        cp.wait()
    xb = x_ref[...]
    inv_xn = 1.0 / jnp.maximum(
        jnp.sqrt(jnp.sum(xb * xb, axis=1)), _EPS)
    xb16 = xb.astype(jnp.bfloat16)
    e = e_vmem[...]
    parts = []
    for i in range(_CL // _CHUNK):
        ec = e[i * _CHUNK:(i + 1) * _CHUNK]
        dots = jax.lax.dot_general(
            ec, xb16, (((1,), (1,)), ((), ())),
            preferred_element_type=jnp.float32)
        parts.append(
            jnp.max(dots.reshape(_CHUNK // _L, _L, _BM), axis=1))
    max_t = jnp.concatenate(parts, axis=0)
    out_ref[...] = (max_t * inv_xn[None, :]).T


def kernel(x, category_embeddings):
    e2 = category_embeddings.reshape(_CL, _D)
    ehat = pl.pallas_call(
        _normalize_body,
        grid=(4,),
        in_specs=[pl.BlockSpec((_CL // 4, _D), lambda i: (i, 0))],
        out_specs=pl.BlockSpec((_CL // 4, _D), lambda i: (i, 0)),
        out_shape=jax.ShapeDtypeStruct((_CL, _D), jnp.bfloat16),
        compiler_params=pltpu.CompilerParams(
            dimension_semantics=("arbitrary",)),
    )(e2)
    out = pl.pallas_call(
        _main_body,
        grid=(_B // _BM,),
        in_specs=[
            pl.BlockSpec((_BM, _D), lambda i: (i, 0)),
            pl.BlockSpec(memory_space=pltpu.MemorySpace.HBM),
        ],
        scratch_shapes=[
            pltpu.VMEM((_CL, _D), jnp.bfloat16),
            pltpu.SemaphoreType.DMA,
        ],
        out_specs=pl.BlockSpec((_BM, _C), lambda i: (i, 0)),
        out_shape=jax.ShapeDtypeStruct((_B, _C), jnp.float32),
        compiler_params=pltpu.CompilerParams(
            dimension_semantics=("arbitrary",)),
    )(x, ehat)
    return out


# speedup vs baseline: 1.1155x; 1.1155x over previous
import jax
import jax.numpy as jnp
from jax.experimental import pallas as pl
from jax.experimental.pallas import tpu as pltpu

_B, _C, _L, _D = 4096, 64, 128, 256
_CL = _C * _L
_EPS = 1e-8

_BM = 1024
_CHUNK = 2048


def _main_body(x_ref, e_hbm, out_ref, e_f32, e_vmem, sem):
    @pl.when(pl.program_id(0) == 0)
    def _():
        cp = pltpu.make_async_copy(e_hbm, e_f32, sem)
        cp.start()
        cp.wait()
        for j in range(4):
            ej = e_f32[j * (_CL // 4):(j + 1) * (_CL // 4), :]
            norm = jnp.sqrt(jnp.sum(ej * ej, axis=1, keepdims=True))
            inv = 1.0 / jnp.maximum(norm, _EPS)
            e_vmem[j * (_CL // 4):(j + 1) * (_CL // 4), :] = (
                ej * inv).astype(jnp.bfloat16)

    xb = x_ref[...]
    inv_xn = 1.0 / jnp.maximum(
        jnp.sqrt(jnp.sum(xb * xb, axis=1)), _EPS)
    xb16 = xb.astype(jnp.bfloat16)
    e = e_vmem[...]
    parts = []
    for i in range(_CL // _CHUNK):
        ec = e[i * _CHUNK:(i + 1) * _CHUNK]
        dots = jax.lax.dot_general(
            ec, xb16, (((1,), (1,)), ((), ())),
            preferred_element_type=jnp.float32)
        parts.append(
            jnp.max(dots.reshape(_CHUNK // _L, _L, _BM), axis=1))
    max_t = jnp.concatenate(parts, axis=0)
    out_ref[...] = (max_t * inv_xn[None, :]).T


def kernel(x, category_embeddings):
    e2 = category_embeddings.reshape(_CL, _D)
    out = pl.pallas_call(
        _main_body,
        grid=(_B // _BM,),
        in_specs=[
            pl.BlockSpec((_BM, _D), lambda i: (i, 0)),
            pl.BlockSpec(memory_space=pltpu.MemorySpace.HBM),
        ],
        scratch_shapes=[
            pltpu.VMEM((_CL, _D), jnp.float32),
            pltpu.VMEM((_CL, _D), jnp.bfloat16),
            pltpu.SemaphoreType.DMA,
        ],
        out_specs=pl.BlockSpec((_BM, _C), lambda i: (i, 0)),
        out_shape=jax.ShapeDtypeStruct((_B, _C), jnp.float32),
        compiler_params=pltpu.CompilerParams(
            dimension_semantics=("arbitrary",)),
    )(x, e2)
    return out


# speedup vs baseline: 1.1719x; 1.0505x over previous
import jax
import jax.numpy as jnp
from jax.experimental import pallas as pl
from jax.experimental.pallas import tpu as pltpu

_B, _C, _L, _D = 4096, 64, 128, 256
_CL = _C * _L
_EPS = 1e-8

_BM = 1024
_CHUNK = 2048


def _main_body(x_ref, e_hbm, out_ref, e_f32, e_vmem, sem):
    @pl.when(pl.program_id(0) == 0)
    def _():
        q = _CL // 4
        cps = [pltpu.make_async_copy(
            e_hbm.at[pl.ds(j * q, q), :],
            e_f32.at[pl.ds(j * q, q), :],
            sem.at[j]) for j in range(4)]
        for cp in cps:
            cp.start()
        for j in range(4):
            cps[j].wait()
            ej = e_f32[j * q:(j + 1) * q, :]
            norm = jnp.sqrt(jnp.sum(ej * ej, axis=1, keepdims=True))
            inv = 1.0 / jnp.maximum(norm, _EPS)
            e_vmem[j * q:(j + 1) * q, :] = (ej * inv).astype(jnp.bfloat16)

    xb = x_ref[...]
    inv_xn = 1.0 / jnp.maximum(
        jnp.sqrt(jnp.sum(xb * xb, axis=1)), _EPS)
    xb16 = xb.astype(jnp.bfloat16)
    e = e_vmem[...]
    parts = []
    for i in range(_CL // _CHUNK):
        ec = e[i * _CHUNK:(i + 1) * _CHUNK]
        dots = jax.lax.dot_general(
            ec, xb16, (((1,), (1,)), ((), ())),
            preferred_element_type=jnp.float32)
        parts.append(
            jnp.max(dots.reshape(_CHUNK // _L, _L, _BM), axis=1))
    max_t = jnp.concatenate(parts, axis=0)
    out_ref[...] = (max_t * inv_xn[None, :]).T


def kernel(x, category_embeddings):
    e2 = category_embeddings.reshape(_CL, _D)
    out = pl.pallas_call(
        _main_body,
        grid=(_B // _BM,),
        in_specs=[
            pl.BlockSpec((_BM, _D), lambda i: (i, 0)),
            pl.BlockSpec(memory_space=pltpu.MemorySpace.HBM),
        ],
        scratch_shapes=[
            pltpu.VMEM((_CL, _D), jnp.float32),
            pltpu.VMEM((_CL, _D), jnp.bfloat16),
            pltpu.SemaphoreType.DMA((4,)),
        ],
        out_specs=pl.BlockSpec((_BM, _C), lambda i: (i, 0)),
        out_shape=jax.ShapeDtypeStruct((_B, _C), jnp.float32),
        compiler_params=pltpu.CompilerParams(
            dimension_semantics=("arbitrary",)),
    )(x, e2)
    return out


# speedup vs baseline: 1.1789x; 1.0060x over previous
import jax
import jax.numpy as jnp
from jax.experimental import pallas as pl
from jax.experimental.pallas import tpu as pltpu

_B, _C, _L, _D = 4096, 64, 128, 256
_CL = _C * _L
_EPS = 1e-8

_BM = 1024
_CHUNK = 1024


def _main_body(x_ref, e_hbm, out_ref, e_f32, e_vmem, sem):
    @pl.when(pl.program_id(0) == 0)
    def _():
        q = _CL // 4
        cps = [pltpu.make_async_copy(
            e_hbm.at[pl.ds(j * q, q), :],
            e_f32.at[pl.ds(j * q, q), :],
            sem.at[j]) for j in range(4)]
        for cp in cps:
            cp.start()
        for j in range(4):
            cps[j].wait()
            ej = e_f32[j * q:(j + 1) * q, :]
            norm = jnp.sqrt(jnp.sum(ej * ej, axis=1, keepdims=True))
            inv = 1.0 / jnp.maximum(norm, _EPS)
            e_vmem[j * q:(j + 1) * q, :] = (ej * inv).astype(jnp.bfloat16)

    xb = x_ref[...]
    inv_xn = 1.0 / jnp.maximum(
        jnp.sqrt(jnp.sum(xb * xb, axis=1)), _EPS)
    xb16 = xb.astype(jnp.bfloat16)
    e = e_vmem[...]
    parts = []
    for i in range(_CL // _CHUNK):
        ec = e[i * _CHUNK:(i + 1) * _CHUNK]
        dots = jax.lax.dot_general(
            ec, xb16, (((1,), (1,)), ((), ())),
            preferred_element_type=jnp.float32)
        parts.append(
            jnp.max(dots.reshape(_CHUNK // _L, _L, _BM), axis=1))
    max_t = jnp.concatenate(parts, axis=0)
    out_ref[...] = (max_t * inv_xn[None, :]).T


def kernel(x, category_embeddings):
    e2 = category_embeddings.reshape(_CL, _D)
    out = pl.pallas_call(
        _main_body,
        grid=(_B // _BM,),
        in_specs=[
            pl.BlockSpec((_BM, _D), lambda i: (i, 0)),
            pl.BlockSpec(memory_space=pltpu.MemorySpace.HBM),
        ],
        scratch_shapes=[
            pltpu.VMEM((_CL, _D), jnp.float32),
            pltpu.VMEM((_CL, _D), jnp.bfloat16),
            pltpu.SemaphoreType.DMA((4,)),
        ],
        out_specs=pl.BlockSpec((_BM, _C), lambda i: (i, 0)),
        out_shape=jax.ShapeDtypeStruct((_B, _C), jnp.float32),
        compiler_params=pltpu.CompilerParams(
            dimension_semantics=("arbitrary",)),
    )(x, e2)
    return out
